# baseline (speedup 1.0000x reference)
import functools
import math

import jax
import jax.numpy as jnp
from jax.experimental import pallas as pl
from jax.experimental.pallas import tpu as pltpu

F32 = jnp.float32
BF16 = jnp.bfloat16


def _dot(a, b):
    return jnp.dot(a.astype(BF16), b.astype(BF16), preferred_element_type=F32)


def _dot_t(a, b, dims):
    return jax.lax.dot_general(a.astype(BF16), b.astype(BF16),
                               (dims, ((), ())), preferred_element_type=F32)


def _ln_rows(x, g, b, eps=1e-5):
    m = jnp.mean(x, axis=-1, keepdims=True)
    v = jnp.mean((x - m) ** 2, axis=-1, keepdims=True)
    return (x - m) / jnp.sqrt(v + eps) * g + b


def _ln_mm_kernel(x_ref, g_ref, b_ref, w_ref, o_ref, xn_ref):
    @pl.when(pl.program_id(1) == 0)
    def _():
        xn_ref[...] = _ln_rows(x_ref[...], g_ref[...], b_ref[...])
    o_ref[...] = _dot(xn_ref[...], w_ref[...])


def _ln_matmul(x2d, g2, b2, w):
    M, D = x2d.shape
    N = w.shape[1]
    BM = min(1024, M)
    BN = min(1024, N)
    return pl.pallas_call(
        _ln_mm_kernel,
        grid=(M // BM, N // BN),
        in_specs=[
            pl.BlockSpec((BM, D), lambda m, n: (m, 0)),
            pl.BlockSpec((1, D), lambda m, n: (0, 0)),
            pl.BlockSpec((1, D), lambda m, n: (0, 0)),
            pl.BlockSpec((D, BN), lambda m, n: (0, n)),
        ],
        out_specs=pl.BlockSpec((BM, BN), lambda m, n: (m, n)),
        out_shape=jax.ShapeDtypeStruct((M, N), F32),
        scratch_shapes=[pltpu.VMEM((BM, D), F32)],
        compiler_params=pltpu.CompilerParams(
            dimension_semantics=("arbitrary", "arbitrary"),
            vmem_limit_bytes=60 * 1024 * 1024,
        ),
    )(x2d, g2, b2, w)


def _attn_kernel(q_ref, k_ref, v_ref, o_ref, *, scale):
    q = q_ref[0]
    k = k_ref[0]
    v = v_ref[0]
    s = _dot_t(q, k, ((1,), (1,))) * scale
    m = jnp.max(s, axis=-1, keepdims=True)
    e = jnp.exp(s - m)
    p = e / jnp.sum(e, axis=-1, keepdims=True)
    o_ref[0] = _dot_t(p, v, ((1,), (0,)))


def _attention(qkv, B, S, D, H):
    hd = D // H
    BQ = min(512, S)
    SQ = S // BQ
    qkv3 = qkv.reshape(B, S, 3 * D)
    kern = functools.partial(_attn_kernel, scale=1.0 / math.sqrt(hd))
    return pl.pallas_call(
        kern,
        grid=(B, H, SQ),
        in_specs=[
            pl.BlockSpec((1, BQ, hd), lambda b, h, sq: (b, sq, h)),
            pl.BlockSpec((1, S, hd), lambda b, h, sq: (b, 0, H + h)),
            pl.BlockSpec((1, S, hd), lambda b, h, sq: (b, 0, 2 * H + h)),
        ],
        out_specs=pl.BlockSpec((1, BQ, hd), lambda b, h, sq: (b, sq, h)),
        out_shape=jax.ShapeDtypeStruct((B, S, D), F32),
        compiler_params=pltpu.CompilerParams(
            dimension_semantics=("arbitrary", "arbitrary", "arbitrary"),
            vmem_limit_bytes=60 * 1024 * 1024,
        ),
    )(qkv3, qkv3, qkv3)


def _mm_res_kernel(a_ref, w_ref, r_ref, o_ref):
    o_ref[...] = r_ref[...] + _dot(a_ref[...], w_ref[...])


def _matmul_residual(a2d, w, r2d):
    M, D = a2d.shape
    BM = min(256, M)
    return pl.pallas_call(
        _mm_res_kernel,
        grid=(M // BM,),
        in_specs=[
            pl.BlockSpec((BM, D), lambda m: (m, 0)),
            pl.BlockSpec((D, D), lambda m: (0, 0)),
            pl.BlockSpec((BM, D), lambda m: (m, 0)),
        ],
        out_specs=pl.BlockSpec((BM, D), lambda m: (m, 0)),
        out_shape=jax.ShapeDtypeStruct((M, D), F32),
        compiler_params=pltpu.CompilerParams(
            dimension_semantics=("arbitrary",),
            vmem_limit_bytes=60 * 1024 * 1024,
        ),
    )(a2d, w, r2d)


def _ffn_kernel(xp_ref, g_ref, b_ref, w1_ref, b1_ref, w2_ref, b2_ref,
                o_ref, xn_ref):
    MBB, _, TE, D = xp_ref.shape
    @pl.when(pl.program_id(2) == 0)
    def _():
        xp = xp_ref[...].reshape(MBB * TE, D)
        xn_ref[...] = _ln_rows(xp, g_ref[...], b_ref[...])
        o_ref[...] = xp_ref[...] + b2_ref[...]
    h = _dot(xn_ref[...], w1_ref[0])
    h = h + b1_ref[0, 0]
    h = h * 0.5 * (1.0 + jax.lax.erf(h * (1.0 / math.sqrt(2.0))))
    eo = _dot(h, w2_ref[0])
    o_ref[...] += eo.reshape(MBB, 1, TE, D)


def _moe_ffn(xp2d, g2, b2ln, e_w1, e_b1, e_w2, e_b2, B, S, D, E, P):
    TE = S // E
    MBB = 2 if B % 2 == 0 else 1
    MB2 = B // MBB
    BP = min(512, P)
    NP = P // BP
    xp4 = xp2d.reshape(B, E, TE, D)
    b1r = e_b1.reshape(E, NP, 1, BP)
    b2r = e_b2.reshape(E, 1, D)
    comb4 = pl.pallas_call(
        _ffn_kernel,
        grid=(E, MB2, NP),
        in_specs=[
            pl.BlockSpec((MBB, 1, TE, D), lambda e, mb, p: (mb, e, 0, 0)),
            pl.BlockSpec((1, D), lambda e, mb, p: (0, 0)),
            pl.BlockSpec((1, D), lambda e, mb, p: (0, 0)),
            pl.BlockSpec((1, D, BP), lambda e, mb, p: (e, 0, p)),
            pl.BlockSpec((1, 1, 1, BP), lambda e, mb, p: (e, p, 0, 0)),
            pl.BlockSpec((1, BP, D), lambda e, mb, p: (e, p, 0)),
            pl.BlockSpec((1, 1, D), lambda e, mb, p: (e, 0, 0)),
        ],
        out_specs=pl.BlockSpec((MBB, 1, TE, D), lambda e, mb, p: (mb, e, 0, 0)),
        out_shape=jax.ShapeDtypeStruct((B, E, TE, D), F32),
        scratch_shapes=[pltpu.VMEM((MBB * TE, D), F32)],
        compiler_params=pltpu.CompilerParams(
            dimension_semantics=("arbitrary", "arbitrary", "arbitrary"),
            vmem_limit_bytes=60 * 1024 * 1024,
        ),
    )(xp4, g2, b2ln, e_w1, b1r, e_w2, b2r)
    return comb4.reshape(B * S, D)


def kernel(x, pre_ln_g, pre_ln_b, pre_wq, pre_wk, pre_wv, pre_wo,
           moe_ln_g, moe_ln_b, gate_w, e_w1, e_b1, e_w2, e_b2,
           post_ln_g, post_ln_b, post_wq, post_wk, post_wv, post_wo):
    B, S, D = x.shape
    E, _, P = e_w1.shape
    H = 16
    x2d = x.reshape(B * S, D)

    wqkv_pre = jnp.concatenate([pre_wq, pre_wk, pre_wv], axis=1)
    wqkv_post = jnp.concatenate([post_wq, post_wk, post_wv], axis=1)
    g = lambda a: a.reshape(1, D)

    qkv = _ln_matmul(x2d, g(pre_ln_g), g(pre_ln_b), wqkv_pre)
    o = _attention(qkv, B, S, D, H)
    xp2d = _matmul_residual(o.reshape(B * S, D), pre_wo, x2d)

    comb2d = _moe_ffn(xp2d, g(moe_ln_g), g(moe_ln_b),
                      e_w1, e_b1, e_w2, e_b2, B, S, D, E, P)

    qkv2 = _ln_matmul(comb2d, g(post_ln_g), g(post_ln_b), wqkv_post)
    o2 = _attention(qkv2, B, S, D, H)
    out2d = _matmul_residual(o2.reshape(B * S, D), post_wo, comb2d)
    return out2d.reshape(B, S, D)


# speedup vs baseline: 1.0803x; 1.0803x over previous
import functools
import math

import jax
import jax.numpy as jnp
from jax.experimental import pallas as pl
from jax.experimental.pallas import tpu as pltpu

F32 = jnp.float32
BF16 = jnp.bfloat16


def _dot(a, b):
    return jnp.dot(a.astype(BF16), b.astype(BF16), preferred_element_type=F32)


def _dot_t(a, b, dims):
    return jax.lax.dot_general(a.astype(BF16), b.astype(BF16),
                               (dims, ((), ())), preferred_element_type=F32)


def _ln_rows(x, g, b, eps=1e-5):
    m = jnp.mean(x, axis=-1, keepdims=True)
    v = jnp.mean((x - m) ** 2, axis=-1, keepdims=True)
    return (x - m) / jnp.sqrt(v + eps) * g + b


def _ln_mm_kernel(x_ref, g_ref, b_ref, w_ref, o_ref, xn_ref):
    @pl.when(pl.program_id(1) == 0)
    def _():
        xn_ref[...] = _ln_rows(x_ref[...], g_ref[...], b_ref[...]).astype(BF16)
    o_ref[...] = _dot(xn_ref[...], w_ref[...]).astype(BF16)


def _ln_matmul(x2d, g2, b2, w):
    M, D = x2d.shape
    N = w.shape[1]
    BM = min(1024, M)
    BN = min(1024, N)
    return pl.pallas_call(
        _ln_mm_kernel,
        grid=(M // BM, N // BN),
        in_specs=[
            pl.BlockSpec((BM, D), lambda m, n: (m, 0)),
            pl.BlockSpec((1, D), lambda m, n: (0, 0)),
            pl.BlockSpec((1, D), lambda m, n: (0, 0)),
            pl.BlockSpec((D, BN), lambda m, n: (0, n)),
        ],
        out_specs=pl.BlockSpec((BM, BN), lambda m, n: (m, n)),
        out_shape=jax.ShapeDtypeStruct((M, N), BF16),
        scratch_shapes=[pltpu.VMEM((BM, D), BF16)],
        compiler_params=pltpu.CompilerParams(
            dimension_semantics=("arbitrary", "arbitrary"),
            vmem_limit_bytes=60 * 1024 * 1024,
        ),
    )(x2d, g2, b2, w)


def _attn_kernel(q_ref, k_ref, v_ref, o_ref, *, scale):
    q = q_ref[0]
    k = k_ref[0]
    v = v_ref[0]
    s = _dot_t(q, k, ((1,), (1,)))
    m = jnp.max(s, axis=-1, keepdims=True)
    e = jnp.exp((s - m) * scale).astype(BF16)
    r = 1.0 / jnp.sum(e, axis=-1, keepdims=True, dtype=F32)
    o = _dot_t(e, v, ((1,), (0,)))
    o_ref[0] = (o * r).astype(BF16)


def _attention(qkv, B, S, D, H):
    hd = D // H
    BQ = min(512, S)
    SQ = S // BQ
    qkv3 = qkv.reshape(B, S, 3 * D)
    kern = functools.partial(_attn_kernel, scale=1.0 / math.sqrt(hd))
    return pl.pallas_call(
        kern,
        grid=(B, H, SQ),
        in_specs=[
            pl.BlockSpec((1, BQ, hd), lambda b, h, sq: (b, sq, h)),
            pl.BlockSpec((1, S, hd), lambda b, h, sq: (b, 0, H + h)),
            pl.BlockSpec((1, S, hd), lambda b, h, sq: (b, 0, 2 * H + h)),
        ],
        out_specs=pl.BlockSpec((1, BQ, hd), lambda b, h, sq: (b, sq, h)),
        out_shape=jax.ShapeDtypeStruct((B, S, D), BF16),
        compiler_params=pltpu.CompilerParams(
            dimension_semantics=("arbitrary", "arbitrary", "arbitrary"),
            vmem_limit_bytes=60 * 1024 * 1024,
        ),
    )(qkv3, qkv3, qkv3)


def _mm_res_kernel(a_ref, w_ref, r_ref, o_ref, wb_ref):
    @pl.when(pl.program_id(0) == 0)
    def _():
        wb_ref[...] = w_ref[...].astype(BF16)
    o_ref[...] = r_ref[...] + _dot(a_ref[...], wb_ref[...])


def _matmul_residual(a2d, w, r2d):
    M, D = a2d.shape
    BM = min(256, M)
    return pl.pallas_call(
        _mm_res_kernel,
        grid=(M // BM,),
        in_specs=[
            pl.BlockSpec((BM, D), lambda m: (m, 0)),
            pl.BlockSpec((D, D), lambda m: (0, 0)),
            pl.BlockSpec((BM, D), lambda m: (m, 0)),
        ],
        out_specs=pl.BlockSpec((BM, D), lambda m: (m, 0)),
        out_shape=jax.ShapeDtypeStruct((M, D), F32),
        scratch_shapes=[pltpu.VMEM((D, D), BF16)],
        compiler_params=pltpu.CompilerParams(
            dimension_semantics=("arbitrary",),
            vmem_limit_bytes=60 * 1024 * 1024,
        ),
    )(a2d, w, r2d)


def _ffn_kernel(xp_ref, g_ref, b_ref, w1_ref, b1_ref, w2_ref, b2_ref,
                o_ref, xn_ref):
    MBB, _, TE, D = xp_ref.shape
    @pl.when(pl.program_id(2) == 0)
    def _():
        xp = xp_ref[...].reshape(MBB * TE, D)
        xn_ref[...] = _ln_rows(xp, g_ref[...], b_ref[...]).astype(BF16)
        o_ref[...] = xp_ref[...] + b2_ref[...]
    h = _dot(xn_ref[...], w1_ref[0])
    h = h + b1_ref[0, 0]
    h = (h * 0.5 * (1.0 + jax.lax.erf(h * (1.0 / math.sqrt(2.0))))).astype(BF16)
    eo = _dot(h, w2_ref[0])
    o_ref[...] += eo.reshape(MBB, 1, TE, D)


def _moe_ffn(xp2d, g2, b2ln, e_w1, e_b1, e_w2, e_b2, B, S, D, E, P):
    TE = S // E
    MBB = B
    MB2 = B // MBB
    BP = min(512, P)
    NP = P // BP
    xp4 = xp2d.reshape(B, E, TE, D)
    b1r = e_b1.reshape(E, NP, 1, BP)
    b2r = e_b2.reshape(E, 1, D)
    comb4 = pl.pallas_call(
        _ffn_kernel,
        grid=(E, MB2, NP),
        in_specs=[
            pl.BlockSpec((MBB, 1, TE, D), lambda e, mb, p: (mb, e, 0, 0)),
            pl.BlockSpec((1, D), lambda e, mb, p: (0, 0)),
            pl.BlockSpec((1, D), lambda e, mb, p: (0, 0)),
            pl.BlockSpec((1, D, BP), lambda e, mb, p: (e, 0, p)),
            pl.BlockSpec((1, 1, 1, BP), lambda e, mb, p: (e, p, 0, 0)),
            pl.BlockSpec((1, BP, D), lambda e, mb, p: (e, p, 0)),
            pl.BlockSpec((1, 1, D), lambda e, mb, p: (e, 0, 0)),
        ],
        out_specs=pl.BlockSpec((MBB, 1, TE, D), lambda e, mb, p: (mb, e, 0, 0)),
        out_shape=jax.ShapeDtypeStruct((B, E, TE, D), F32),
        scratch_shapes=[pltpu.VMEM((MBB * TE, D), BF16)],
        compiler_params=pltpu.CompilerParams(
            dimension_semantics=("arbitrary", "arbitrary", "arbitrary"),
            vmem_limit_bytes=60 * 1024 * 1024,
        ),
    )(xp4, g2, b2ln, e_w1, b1r, e_w2, b2r)
    return comb4.reshape(B * S, D)


def kernel(x, pre_ln_g, pre_ln_b, pre_wq, pre_wk, pre_wv, pre_wo,
           moe_ln_g, moe_ln_b, gate_w, e_w1, e_b1, e_w2, e_b2,
           post_ln_g, post_ln_b, post_wq, post_wk, post_wv, post_wo):
    B, S, D = x.shape
    E, _, P = e_w1.shape
    H = 16
    x2d = x.reshape(B * S, D)

    wqkv_pre = jnp.concatenate([pre_wq, pre_wk, pre_wv], axis=1)
    wqkv_post = jnp.concatenate([post_wq, post_wk, post_wv], axis=1)
    g = lambda a: a.reshape(1, D)

    qkv = _ln_matmul(x2d, g(pre_ln_g), g(pre_ln_b), wqkv_pre)
    o = _attention(qkv, B, S, D, H)
    xp2d = _matmul_residual(o.reshape(B * S, D), pre_wo, x2d)

    comb2d = _moe_ffn(xp2d, g(moe_ln_g), g(moe_ln_b),
                      e_w1, e_b1, e_w2, e_b2, B, S, D, E, P)

    qkv2 = _ln_matmul(comb2d, g(post_ln_g), g(post_ln_b), wqkv_post)
    o2 = _attention(qkv2, B, S, D, H)
    out2d = _matmul_residual(o2.reshape(B * S, D), post_wo, comb2d)
    return out2d.reshape(B, S, D)


# speedup vs baseline: 1.1995x; 1.1103x over previous
import functools
import math

import jax
import jax.numpy as jnp
from jax.experimental import pallas as pl
from jax.experimental.pallas import tpu as pltpu

F32 = jnp.float32
BF16 = jnp.bfloat16


def _dot(a, b):
    return jnp.dot(a.astype(BF16), b.astype(BF16), preferred_element_type=F32)


def _dot_t(a, b, dims):
    return jax.lax.dot_general(a.astype(BF16), b.astype(BF16),
                               (dims, ((), ())), preferred_element_type=F32)


def _ln_rows(x, g, b, eps=1e-5):
    m = jnp.mean(x, axis=-1, keepdims=True)
    v = jnp.mean((x - m) ** 2, axis=-1, keepdims=True)
    return (x - m) / jnp.sqrt(v + eps) * g + b


def _ln_mm_kernel(x_ref, g_ref, b_ref, w_ref, o_ref, xn_ref):
    @pl.when(pl.program_id(1) == 0)
    def _():
        xn_ref[...] = _ln_rows(x_ref[...], g_ref[...], b_ref[...]).astype(BF16)
    o_ref[...] = _dot(xn_ref[...], w_ref[...]).astype(BF16)


def _ln_matmul(x2d, g2, b2, w):
    M, D = x2d.shape
    N = w.shape[1]
    BM = min(1024, M)
    BN = min(1024, N)
    return pl.pallas_call(
        _ln_mm_kernel,
        grid=(M // BM, N // BN),
        in_specs=[
            pl.BlockSpec((BM, D), lambda m, n: (m, 0)),
            pl.BlockSpec((1, D), lambda m, n: (0, 0)),
            pl.BlockSpec((1, D), lambda m, n: (0, 0)),
            pl.BlockSpec((D, BN), lambda m, n: (0, n)),
        ],
        out_specs=pl.BlockSpec((BM, BN), lambda m, n: (m, n)),
        out_shape=jax.ShapeDtypeStruct((M, N), BF16),
        scratch_shapes=[pltpu.VMEM((BM, D), BF16)],
        compiler_params=pltpu.CompilerParams(
            dimension_semantics=("arbitrary", "arbitrary"),
            vmem_limit_bytes=60 * 1024 * 1024,
        ),
    )(x2d, g2, b2, w)


def _attn_kernel(q_ref, k_ref, v_ref, o_ref, va_ref, *, c, hd):
    @pl.when(pl.program_id(2) == 0)
    def _():
        va_ref[:, :hd] = v_ref[0].astype(BF16)
        va_ref[:, hd:] = jnp.ones_like(v_ref[0], dtype=BF16)
    q = q_ref[0]
    k = k_ref[0]
    s = _dot_t(q, k, ((1,), (1,)))
    m = jnp.max(s, axis=-1, keepdims=True)
    e = jnp.exp2((s - m) * c).astype(BF16)
    oa = _dot_t(e, va_ref[...], ((1,), (0,)))
    o_ref[0] = (oa[:, :hd] / oa[:, hd:]).astype(BF16)


def _attention(qkv, B, S, D, H):
    hd = D // H
    BQ = min(512, S)
    SQ = S // BQ
    qkv3 = qkv.reshape(B, S, 3 * D)
    kern = functools.partial(_attn_kernel, c=math.log2(math.e) / math.sqrt(hd),
                             hd=hd)
    return pl.pallas_call(
        kern,
        grid=(B, H, SQ),
        in_specs=[
            pl.BlockSpec((1, BQ, hd), lambda b, h, sq: (b, sq, h)),
            pl.BlockSpec((1, S, hd), lambda b, h, sq: (b, 0, H + h)),
            pl.BlockSpec((1, S, hd), lambda b, h, sq: (b, 0, 2 * H + h)),
        ],
        out_specs=pl.BlockSpec((1, BQ, hd), lambda b, h, sq: (b, sq, h)),
        out_shape=jax.ShapeDtypeStruct((B, S, D), BF16),
        scratch_shapes=[pltpu.VMEM((S, 2 * hd), BF16)],
        compiler_params=pltpu.CompilerParams(
            dimension_semantics=("arbitrary", "arbitrary", "arbitrary"),
            vmem_limit_bytes=60 * 1024 * 1024,
        ),
    )(qkv3, qkv3, qkv3)


def _mm_res_kernel(a_ref, w_ref, r_ref, o_ref, wb_ref):
    @pl.when(pl.program_id(0) == 0)
    def _():
        wb_ref[...] = w_ref[...].astype(BF16)
    o_ref[...] = r_ref[...] + _dot(a_ref[...], wb_ref[...])


def _matmul_residual(a2d, w, r2d):
    M, D = a2d.shape
    BM = min(256, M)
    return pl.pallas_call(
        _mm_res_kernel,
        grid=(M // BM,),
        in_specs=[
            pl.BlockSpec((BM, D), lambda m: (m, 0)),
            pl.BlockSpec((D, D), lambda m: (0, 0)),
            pl.BlockSpec((BM, D), lambda m: (m, 0)),
        ],
        out_specs=pl.BlockSpec((BM, D), lambda m: (m, 0)),
        out_shape=jax.ShapeDtypeStruct((M, D), F32),
        scratch_shapes=[pltpu.VMEM((D, D), BF16)],
        compiler_params=pltpu.CompilerParams(
            dimension_semantics=("arbitrary",),
            vmem_limit_bytes=60 * 1024 * 1024,
        ),
    )(a2d, w, r2d)


def _ffn_kernel(xp_ref, g_ref, b_ref, w1_ref, b1_ref, w2_ref, b2_ref,
                o_ref, xn_ref):
    MBB, _, TE, D = xp_ref.shape
    @pl.when(pl.program_id(2) == 0)
    def _():
        xp = xp_ref[...].reshape(MBB * TE, D)
        xn_ref[...] = _ln_rows(xp, g_ref[...], b_ref[...]).astype(BF16)
        o_ref[...] = xp_ref[...] + b2_ref[...]
    h = _dot(xn_ref[...], w1_ref[0])
    h = h + b1_ref[0, 0]
    h = (h * 0.5 * (1.0 + jax.lax.erf(h * (1.0 / math.sqrt(2.0))))).astype(BF16)
    eo = _dot(h, w2_ref[0])
    o_ref[...] += eo.reshape(MBB, 1, TE, D)


def _moe_ffn(xp2d, g2, b2ln, e_w1, e_b1, e_w2, e_b2, B, S, D, E, P):
    TE = S // E
    MBB = B
    MB2 = B // MBB
    BP = min(512, P)
    NP = P // BP
    xp4 = xp2d.reshape(B, E, TE, D)
    b1r = e_b1.reshape(E, NP, 1, BP)
    b2r = e_b2.reshape(E, 1, D)
    comb4 = pl.pallas_call(
        _ffn_kernel,
        grid=(E, MB2, NP),
        in_specs=[
            pl.BlockSpec((MBB, 1, TE, D), lambda e, mb, p: (mb, e, 0, 0)),
            pl.BlockSpec((1, D), lambda e, mb, p: (0, 0)),
            pl.BlockSpec((1, D), lambda e, mb, p: (0, 0)),
            pl.BlockSpec((1, D, BP), lambda e, mb, p: (e, 0, p)),
            pl.BlockSpec((1, 1, 1, BP), lambda e, mb, p: (e, p, 0, 0)),
            pl.BlockSpec((1, BP, D), lambda e, mb, p: (e, p, 0)),
            pl.BlockSpec((1, 1, D), lambda e, mb, p: (e, 0, 0)),
        ],
        out_specs=pl.BlockSpec((MBB, 1, TE, D), lambda e, mb, p: (mb, e, 0, 0)),
        out_shape=jax.ShapeDtypeStruct((B, E, TE, D), F32),
        scratch_shapes=[pltpu.VMEM((MBB * TE, D), BF16)],
        compiler_params=pltpu.CompilerParams(
            dimension_semantics=("arbitrary", "arbitrary", "arbitrary"),
            vmem_limit_bytes=60 * 1024 * 1024,
        ),
    )(xp4, g2, b2ln, e_w1, b1r, e_w2, b2r)
    return comb4.reshape(B * S, D)


def kernel(x, pre_ln_g, pre_ln_b, pre_wq, pre_wk, pre_wv, pre_wo,
           moe_ln_g, moe_ln_b, gate_w, e_w1, e_b1, e_w2, e_b2,
           post_ln_g, post_ln_b, post_wq, post_wk, post_wv, post_wo):
    B, S, D = x.shape
    E, _, P = e_w1.shape
    H = 16
    x2d = x.reshape(B * S, D)

    wqkv_pre = jnp.concatenate([pre_wq, pre_wk, pre_wv], axis=1)
    wqkv_post = jnp.concatenate([post_wq, post_wk, post_wv], axis=1)
    g = lambda a: a.reshape(1, D)

    qkv = _ln_matmul(x2d, g(pre_ln_g), g(pre_ln_b), wqkv_pre)
    o = _attention(qkv, B, S, D, H)
    xp2d = _matmul_residual(o.reshape(B * S, D), pre_wo, x2d)

    comb2d = _moe_ffn(xp2d, g(moe_ln_g), g(moe_ln_b),
                      e_w1, e_b1, e_w2, e_b2, B, S, D, E, P)

    qkv2 = _ln_matmul(comb2d, g(post_ln_g), g(post_ln_b), wqkv_post)
    o2 = _attention(qkv2, B, S, D, H)
    out2d = _matmul_residual(o2.reshape(B * S, D), post_wo, comb2d)
    return out2d.reshape(B, S, D)


# speedup vs baseline: 1.3471x; 1.1231x over previous
import functools
import math

import jax
import jax.numpy as jnp
from jax.experimental import pallas as pl
from jax.experimental.pallas import tpu as pltpu

F32 = jnp.float32
BF16 = jnp.bfloat16


def _dot(a, b):
    return jnp.dot(a.astype(BF16), b.astype(BF16), preferred_element_type=F32)


def _dot_t(a, b, dims):
    return jax.lax.dot_general(a.astype(BF16), b.astype(BF16),
                               (dims, ((), ())), preferred_element_type=F32)


def _ln_rows(x, g, b, eps=1e-5):
    m = jnp.mean(x, axis=-1, keepdims=True)
    v = jnp.mean((x - m) ** 2, axis=-1, keepdims=True)
    return (x - m) / jnp.sqrt(v + eps) * g + b


def _ln_mm_kernel(x_ref, g_ref, b_ref, w_ref, o_ref, xn_ref):
    @pl.when(pl.program_id(1) == 0)
    def _():
        xn_ref[...] = _ln_rows(x_ref[...], g_ref[...], b_ref[...]).astype(BF16)
    o_ref[...] = _dot(xn_ref[...], w_ref[...]).astype(BF16)


def _ln_matmul(x2d, g2, b2, w):
    M, D = x2d.shape
    N = w.shape[1]
    BM = min(1024, M)
    BN = min(1024, N)
    return pl.pallas_call(
        _ln_mm_kernel,
        grid=(M // BM, N // BN),
        in_specs=[
            pl.BlockSpec((BM, D), lambda m, n: (m, 0)),
            pl.BlockSpec((1, D), lambda m, n: (0, 0)),
            pl.BlockSpec((1, D), lambda m, n: (0, 0)),
            pl.BlockSpec((D, BN), lambda m, n: (0, n)),
        ],
        out_specs=pl.BlockSpec((BM, BN), lambda m, n: (m, n)),
        out_shape=jax.ShapeDtypeStruct((M, N), BF16),
        scratch_shapes=[pltpu.VMEM((BM, D), BF16)],
        compiler_params=pltpu.CompilerParams(
            dimension_semantics=("arbitrary", "arbitrary"),
            vmem_limit_bytes=60 * 1024 * 1024,
        ),
    )(x2d, g2, b2, w)


def _attn_kernel(q_ref, k_ref, v_ref, o_ref, va_ref, *, c, hd):
    @pl.when(pl.program_id(2) == 0)
    def _():
        va_ref[:, :hd] = v_ref[0].astype(BF16)
        va_ref[:, hd:] = jnp.ones_like(v_ref[0], dtype=BF16)
    k = k_ref[0]
    va = va_ref[...]
    BQ = q_ref.shape[1]
    NC = 4
    CH = BQ // NC
    outs = []
    for i in range(NC):
        qi = q_ref[0, i * CH:(i + 1) * CH, :]
        si = _dot_t(qi, k, ((1,), (1,)))
        mi = jnp.max(si, axis=-1, keepdims=True)
        ei = jnp.exp2((si - mi) * c).astype(BF16)
        outs.append(_dot_t(ei, va, ((1,), (0,))))
    oa = jnp.concatenate(outs, axis=0)
    o_ref[0] = (oa[:, :hd] / oa[:, hd:]).astype(BF16)


def _attention(qkv, B, S, D, H):
    hd = D // H
    BQ = min(512, S)
    SQ = S // BQ
    qkv3 = qkv.reshape(B, S, 3 * D)
    kern = functools.partial(_attn_kernel, c=math.log2(math.e) / math.sqrt(hd),
                             hd=hd)
    return pl.pallas_call(
        kern,
        grid=(B, H, SQ),
        in_specs=[
            pl.BlockSpec((1, BQ, hd), lambda b, h, sq: (b, sq, h)),
            pl.BlockSpec((1, S, hd), lambda b, h, sq: (b, 0, H + h)),
            pl.BlockSpec((1, S, hd), lambda b, h, sq: (b, 0, 2 * H + h)),
        ],
        out_specs=pl.BlockSpec((1, BQ, hd), lambda b, h, sq: (b, sq, h)),
        out_shape=jax.ShapeDtypeStruct((B, S, D), BF16),
        scratch_shapes=[pltpu.VMEM((S, 2 * hd), BF16)],
        compiler_params=pltpu.CompilerParams(
            dimension_semantics=("arbitrary", "arbitrary", "arbitrary"),
            vmem_limit_bytes=60 * 1024 * 1024,
        ),
    )(qkv3, qkv3, qkv3)


def _mm_res_kernel(a_ref, w_ref, r_ref, o_ref, wb_ref):
    @pl.when(pl.program_id(0) == 0)
    def _():
        wb_ref[...] = w_ref[...].astype(BF16)
    o_ref[...] = r_ref[...] + _dot(a_ref[...], wb_ref[...])


def _matmul_residual(a2d, w, r2d):
    M, D = a2d.shape
    BM = min(256, M)
    return pl.pallas_call(
        _mm_res_kernel,
        grid=(M // BM,),
        in_specs=[
            pl.BlockSpec((BM, D), lambda m: (m, 0)),
            pl.BlockSpec((D, D), lambda m: (0, 0)),
            pl.BlockSpec((BM, D), lambda m: (m, 0)),
        ],
        out_specs=pl.BlockSpec((BM, D), lambda m: (m, 0)),
        out_shape=jax.ShapeDtypeStruct((M, D), F32),
        scratch_shapes=[pltpu.VMEM((D, D), BF16)],
        compiler_params=pltpu.CompilerParams(
            dimension_semantics=("arbitrary",),
            vmem_limit_bytes=60 * 1024 * 1024,
        ),
    )(a2d, w, r2d)


def _ffn_kernel(xp_ref, g_ref, b_ref, w1_ref, b1_ref, w2_ref, b2_ref,
                o_ref, xn_ref):
    MBB, _, TE, D = xp_ref.shape
    @pl.when(pl.program_id(2) == 0)
    def _():
        xp = xp_ref[...].reshape(MBB * TE, D)
        xn_ref[...] = _ln_rows(xp, g_ref[...], b_ref[...]).astype(BF16)
        o_ref[...] = xp_ref[...] + b2_ref[...]
    h = _dot(xn_ref[...], w1_ref[0])
    h = h + b1_ref[0, 0]
    h = (h * 0.5 * (1.0 + jax.lax.erf(h * (1.0 / math.sqrt(2.0))))).astype(BF16)
    eo = _dot(h, w2_ref[0])
    o_ref[...] += eo.reshape(MBB, 1, TE, D)


def _moe_ffn(xp2d, g2, b2ln, e_w1, e_b1, e_w2, e_b2, B, S, D, E, P):
    TE = S // E
    MBB = B
    MB2 = B // MBB
    BP = min(512, P)
    NP = P // BP
    xp4 = xp2d.reshape(B, E, TE, D)
    b1r = e_b1.reshape(E, NP, 1, BP)
    b2r = e_b2.reshape(E, 1, D)
    comb4 = pl.pallas_call(
        _ffn_kernel,
        grid=(E, MB2, NP),
        in_specs=[
            pl.BlockSpec((MBB, 1, TE, D), lambda e, mb, p: (mb, e, 0, 0)),
            pl.BlockSpec((1, D), lambda e, mb, p: (0, 0)),
            pl.BlockSpec((1, D), lambda e, mb, p: (0, 0)),
            pl.BlockSpec((1, D, BP), lambda e, mb, p: (e, 0, p)),
            pl.BlockSpec((1, 1, 1, BP), lambda e, mb, p: (e, p, 0, 0)),
            pl.BlockSpec((1, BP, D), lambda e, mb, p: (e, p, 0)),
            pl.BlockSpec((1, 1, D), lambda e, mb, p: (e, 0, 0)),
        ],
        out_specs=pl.BlockSpec((MBB, 1, TE, D), lambda e, mb, p: (mb, e, 0, 0)),
        out_shape=jax.ShapeDtypeStruct((B, E, TE, D), F32),
        scratch_shapes=[pltpu.VMEM((MBB * TE, D), BF16)],
        compiler_params=pltpu.CompilerParams(
            dimension_semantics=("arbitrary", "arbitrary", "arbitrary"),
            vmem_limit_bytes=60 * 1024 * 1024,
        ),
    )(xp4, g2, b2ln, e_w1, b1r, e_w2, b2r)
    return comb4.reshape(B * S, D)


def kernel(x, pre_ln_g, pre_ln_b, pre_wq, pre_wk, pre_wv, pre_wo,
           moe_ln_g, moe_ln_b, gate_w, e_w1, e_b1, e_w2, e_b2,
           post_ln_g, post_ln_b, post_wq, post_wk, post_wv, post_wo):
    B, S, D = x.shape
    E, _, P = e_w1.shape
    H = 16
    x2d = x.reshape(B * S, D)

    wqkv_pre = jnp.concatenate([pre_wq, pre_wk, pre_wv], axis=1)
    wqkv_post = jnp.concatenate([post_wq, post_wk, post_wv], axis=1)
    g = lambda a: a.reshape(1, D)

    qkv = _ln_matmul(x2d, g(pre_ln_g), g(pre_ln_b), wqkv_pre)
    o = _attention(qkv, B, S, D, H)
    xp2d = _matmul_residual(o.reshape(B * S, D), pre_wo, x2d)

    comb2d = _moe_ffn(xp2d, g(moe_ln_g), g(moe_ln_b),
                      e_w1, e_b1, e_w2, e_b2, B, S, D, E, P)

    qkv2 = _ln_matmul(comb2d, g(post_ln_g), g(post_ln_b), wqkv_post)
    o2 = _attention(qkv2, B, S, D, H)
    out2d = _matmul_residual(o2.reshape(B * S, D), post_wo, comb2d)
    return out2d.reshape(B, S, D)


# speedup vs baseline: 1.3508x; 1.0027x over previous
import functools
import math

import jax
import jax.numpy as jnp
from jax.experimental import pallas as pl
from jax.experimental.pallas import tpu as pltpu

F32 = jnp.float32
BF16 = jnp.bfloat16


def _dot(a, b):
    return jnp.dot(a.astype(BF16), b.astype(BF16), preferred_element_type=F32)


def _dot_t(a, b, dims):
    return jax.lax.dot_general(a.astype(BF16), b.astype(BF16),
                               (dims, ((), ())), preferred_element_type=F32)


def _ln_rows(x, g, b, eps=1e-5):
    m = jnp.mean(x, axis=-1, keepdims=True)
    v = jnp.mean((x - m) ** 2, axis=-1, keepdims=True)
    return (x - m) / jnp.sqrt(v + eps) * g + b


def _ln_mm_kernel(x_ref, g_ref, b_ref, w_ref, o_ref, xn_ref):
    @pl.when(pl.program_id(1) == 0)
    def _():
        xn_ref[...] = _ln_rows(x_ref[...], g_ref[...], b_ref[...]).astype(BF16)
    o_ref[...] = _dot(xn_ref[...], w_ref[...]).astype(BF16)


def _ln_matmul(x2d, g2, b2, w):
    M, D = x2d.shape
    N = w.shape[1]
    BM = min(1024, M)
    BN = min(1024, N)
    return pl.pallas_call(
        _ln_mm_kernel,
        grid=(M // BM, N // BN),
        in_specs=[
            pl.BlockSpec((BM, D), lambda m, n: (m, 0)),
            pl.BlockSpec((1, D), lambda m, n: (0, 0)),
            pl.BlockSpec((1, D), lambda m, n: (0, 0)),
            pl.BlockSpec((D, BN), lambda m, n: (0, n)),
        ],
        out_specs=pl.BlockSpec((BM, BN), lambda m, n: (m, n)),
        out_shape=jax.ShapeDtypeStruct((M, N), BF16),
        scratch_shapes=[pltpu.VMEM((BM, D), BF16)],
        compiler_params=pltpu.CompilerParams(
            dimension_semantics=("arbitrary", "arbitrary"),
            vmem_limit_bytes=60 * 1024 * 1024,
        ),
    )(x2d, g2, b2, w)


def _attn_kernel(q_ref, k_ref, v_ref, o_ref, va_ref, *, c, hd):
    @pl.when(pl.program_id(2) == 0)
    def _():
        va_ref[:, :hd] = v_ref[0].astype(BF16)
        va_ref[:, hd:] = jnp.ones_like(v_ref[0], dtype=BF16)
    k = k_ref[0]
    va = va_ref[...]
    BQ = q_ref.shape[1]
    NC = 4
    CH = BQ // NC
    outs = []
    for i in range(NC):
        qi = q_ref[0, i * CH:(i + 1) * CH, :]
        si = _dot_t(qi, k, ((1,), (1,)))
        mi = jnp.max(si, axis=-1, keepdims=True)
        ei = jnp.exp2((si - mi) * c).astype(BF16)
        outs.append(_dot_t(ei, va, ((1,), (0,))))
    oa = jnp.concatenate(outs, axis=0)
    o_ref[0] = (oa[:, :hd] / oa[:, hd:]).astype(BF16)


def _attention(qkv, B, S, D, H):
    hd = D // H
    BQ = min(512, S)
    SQ = S // BQ
    qkv3 = qkv.reshape(B, S, 3 * D)
    kern = functools.partial(_attn_kernel, c=math.log2(math.e) / math.sqrt(hd),
                             hd=hd)
    return pl.pallas_call(
        kern,
        grid=(B, H, SQ),
        in_specs=[
            pl.BlockSpec((1, BQ, hd), lambda b, h, sq: (b, sq, h)),
            pl.BlockSpec((1, S, hd), lambda b, h, sq: (b, 0, H + h)),
            pl.BlockSpec((1, S, hd), lambda b, h, sq: (b, 0, 2 * H + h)),
        ],
        out_specs=pl.BlockSpec((1, BQ, hd), lambda b, h, sq: (b, sq, h)),
        out_shape=jax.ShapeDtypeStruct((B, S, D), BF16),
        scratch_shapes=[pltpu.VMEM((S, 2 * hd), BF16)],
        compiler_params=pltpu.CompilerParams(
            dimension_semantics=("arbitrary", "arbitrary", "arbitrary"),
            vmem_limit_bytes=60 * 1024 * 1024,
        ),
    )(qkv3, qkv3, qkv3)


def _mm_res_kernel(a_ref, w_ref, r_ref, o_ref):
    o_ref[...] = r_ref[...] + _dot(a_ref[...], w_ref[...])


def _matmul_residual(a2d, w, r2d):
    M, D = a2d.shape
    BM = min(256, M)
    return pl.pallas_call(
        _mm_res_kernel,
        grid=(M // BM,),
        in_specs=[
            pl.BlockSpec((BM, D), lambda m: (m, 0)),
            pl.BlockSpec((D, D), lambda m: (0, 0)),
            pl.BlockSpec((BM, D), lambda m: (m, 0)),
        ],
        out_specs=pl.BlockSpec((BM, D), lambda m: (m, 0)),
        out_shape=jax.ShapeDtypeStruct((M, D), F32),
        compiler_params=pltpu.CompilerParams(
            dimension_semantics=("arbitrary",),
            vmem_limit_bytes=60 * 1024 * 1024,
        ),
    )(a2d, w, r2d)


def _ffn_kernel(xp_ref, g_ref, b_ref, w1_ref, b1_ref, w2_ref, b2_ref,
                o_ref, xn_ref):
    MBB, _, TE, D = xp_ref.shape
    @pl.when(pl.program_id(2) == 0)
    def _():
        xp = xp_ref[...].reshape(MBB * TE, D)
        xn_ref[...] = _ln_rows(xp, g_ref[...], b_ref[...]).astype(BF16)
        o_ref[...] = xp_ref[...] + b2_ref[...]
    xn = xn_ref[...]
    w1 = w1_ref[0]
    w2 = w2_ref[0]
    b1 = b1_ref[0, 0]
    BP = w1.shape[1]
    HB = BP // 2
    inv_sqrt2 = 1.0 / math.sqrt(2.0)
    eos = []
    for lo in (0, HB):
        h = _dot(xn, w1[:, lo:lo + HB]) + b1[:, lo:lo + HB]
        h = (h * 0.5 * (1.0 + jax.lax.erf(h * inv_sqrt2))).astype(BF16)
        eos.append(_dot(h, w2[lo:lo + HB, :]))
    o_ref[...] += (eos[0] + eos[1]).reshape(MBB, 1, TE, D)


def _moe_ffn(xp2d, g2, b2ln, e_w1, e_b1, e_w2, e_b2, B, S, D, E, P):
    TE = S // E
    MBB = B
    MB2 = B // MBB
    BP = min(512, P)
    NP = P // BP
    xp4 = xp2d.reshape(B, E, TE, D)
    b1r = e_b1.reshape(E, NP, 1, BP)
    b2r = e_b2.reshape(E, 1, D)
    comb4 = pl.pallas_call(
        _ffn_kernel,
        grid=(E, MB2, NP),
        in_specs=[
            pl.BlockSpec((MBB, 1, TE, D), lambda e, mb, p: (mb, e, 0, 0)),
            pl.BlockSpec((1, D), lambda e, mb, p: (0, 0)),
            pl.BlockSpec((1, D), lambda e, mb, p: (0, 0)),
            pl.BlockSpec((1, D, BP), lambda e, mb, p: (e, 0, p)),
            pl.BlockSpec((1, 1, 1, BP), lambda e, mb, p: (e, p, 0, 0)),
            pl.BlockSpec((1, BP, D), lambda e, mb, p: (e, p, 0)),
            pl.BlockSpec((1, 1, D), lambda e, mb, p: (e, 0, 0)),
        ],
        out_specs=pl.BlockSpec((MBB, 1, TE, D), lambda e, mb, p: (mb, e, 0, 0)),
        out_shape=jax.ShapeDtypeStruct((B, E, TE, D), F32),
        scratch_shapes=[pltpu.VMEM((MBB * TE, D), BF16)],
        compiler_params=pltpu.CompilerParams(
            dimension_semantics=("arbitrary", "arbitrary", "arbitrary"),
            vmem_limit_bytes=64 * 1024 * 1024,
        ),
    )(xp4, g2, b2ln, e_w1, b1r, e_w2, b2r)
    return comb4.reshape(B * S, D)


def kernel(x, pre_ln_g, pre_ln_b, pre_wq, pre_wk, pre_wv, pre_wo,
           moe_ln_g, moe_ln_b, gate_w, e_w1, e_b1, e_w2, e_b2,
           post_ln_g, post_ln_b, post_wq, post_wk, post_wv, post_wo):
    B, S, D = x.shape
    E, _, P = e_w1.shape
    H = 16
    x2d = x.reshape(B * S, D)

    wqkv_pre = jnp.concatenate([pre_wq, pre_wk, pre_wv], axis=1)
    wqkv_post = jnp.concatenate([post_wq, post_wk, post_wv], axis=1)
    g = lambda a: a.reshape(1, D)

    qkv = _ln_matmul(x2d, g(pre_ln_g), g(pre_ln_b), wqkv_pre)
    o = _attention(qkv, B, S, D, H)
    xp2d = _matmul_residual(o.reshape(B * S, D), pre_wo, x2d)

    comb2d = _moe_ffn(xp2d, g(moe_ln_g), g(moe_ln_b),
                      e_w1, e_b1, e_w2, e_b2, B, S, D, E, P)

    qkv2 = _ln_matmul(comb2d, g(post_ln_g), g(post_ln_b), wqkv_post)
    o2 = _attention(qkv2, B, S, D, H)
    out2d = _matmul_residual(o2.reshape(B * S, D), post_wo, comb2d)
    return out2d.reshape(B, S, D)
